# baseline (speedup 1.0000x reference)
import functools

import jax
import jax.numpy as jnp
from jax import lax
from jax.experimental import pallas as pl
from jax.experimental.pallas import tpu as pltpu
from jax.experimental.pallas import tpu_sc as plsc

_B, _N, _K = 4, 10000, 16
_GROUPS = 8
_PB = 400
_NB = _N // _PB


def _leaky(x):
    return jnp.where(x >= 0, x, 0.1 * x)


def _mm(w, x):
    return lax.dot_general(w, x, (((1,), (0,)), ((), ())))


def _order_map(w):
    return w ^ (((w >> 15) & jnp.uint32(1)) * jnp.uint32(0x7FFF))


def _f16_bits(h):
    u = lax.bitcast_convert_type(h, jnp.uint32)
    sign = (u >> 16) & jnp.uint32(0x8000)
    mag = u & jnp.uint32(0x7FFFFFFF)
    r = (mag + jnp.uint32(0xFFF) + ((mag >> 13) & jnp.uint32(1))) >> 13
    m16 = jnp.maximum(
        lax.convert_element_type(r, jnp.int32) - jnp.int32(0x1C000),
        jnp.int32(0))
    return sign | lax.convert_element_type(m16, jnp.uint32)


def _pack(h):
    u = _f16_bits(h)
    lo = _order_map(u[:32])
    hi = _order_map(u[32:])
    return lax.bitcast_convert_type(lo | (hi << 16), jnp.int32)


def _half_to_f32(w):
    b = _order_map(w)
    sign32 = (b & jnp.uint32(0x8000)) << 16
    mag16 = b & jnp.uint32(0x7FFF)
    mag32 = (mag16 + jnp.uint32(0x1C000)) << 13
    mag32 = jnp.where(mag16 == 0, jnp.uint32(0), mag32)
    return lax.bitcast_convert_type(sign32 | mag32, jnp.float32)


def _unpack(mp):
    u = lax.bitcast_convert_type(mp, jnp.uint32)
    lo = _half_to_f32(u & jnp.uint32(0xFFFF))
    hi = _half_to_f32(u >> 16)
    return lax.concatenate([lo, hi], 0)


def _stage_a(xyz_ref, feat_ref, w1x_ref, w1f_ref, h1_ref):
    h1 = _mm(w1x_ref[...], xyz_ref[0]) + _mm(w1f_ref[...], feat_ref[0])
    h1_ref[0] = _pack(h1)


def _stage_b(m1_ref, xyz_ref, w1x_ref, b1_ref, w2x_ref, w2f_ref, h2_ref):
    f1 = _leaky(_unpack(m1_ref[0]) - _mm(w1x_ref[...], xyz_ref[0])
                + b1_ref[...])
    h2_ref[0] = _pack(_mm(w2x_ref[...], xyz_ref[0]) + _mm(w2f_ref[...], f1))


def _stage_c(m2_ref, xyz_ref, w2x_ref, b2_ref, wm1_ref, bm1_ref, wm2_ref,
             bm2_ref, wl_ref, bl_ref, h_ref, flow_ref):
    f2 = _leaky(_unpack(m2_ref[0]) - _mm(w2x_ref[...], xyz_ref[0])
                + b2_ref[...])
    h1 = _leaky(_mm(wm1_ref[...], f2) + bm1_ref[...])
    h = _leaky(_mm(wm2_ref[...], h1) + bm2_ref[...])
    h_ref[0] = h
    flow_ref[0] = _mm(wl_ref[...], h) + bl_ref[...]


def _batch_spec(c, n):
    return pl.BlockSpec((1, c, n), lambda b: (b, 0, 0))


def _full_spec(*shape):
    return pl.BlockSpec(shape, lambda b: tuple(0 for _ in shape))


def _run_stage_a(xyz8, feat, w1x, w1f):
    return pl.pallas_call(
        _stage_a,
        grid=(_B,),
        in_specs=[_batch_spec(8, _N), _batch_spec(128, _N),
                  _full_spec(64, 8), _full_spec(64, 128)],
        out_specs=_batch_spec(32, _N),
        out_shape=jax.ShapeDtypeStruct((_B, 32, _N), jnp.int32),
    )(xyz8, feat, w1x, w1f)


def _run_stage_b(m1, xyz8, w1x, b1c, w2x, w2f):
    return pl.pallas_call(
        _stage_b,
        grid=(_B,),
        in_specs=[_batch_spec(32, _N), _batch_spec(8, _N),
                  _full_spec(64, 8), _full_spec(64, 1),
                  _full_spec(64, 8), _full_spec(64, 64)],
        out_specs=_batch_spec(32, _N),
        out_shape=jax.ShapeDtypeStruct((_B, 32, _N), jnp.int32),
    )(m1, xyz8, w1x, b1c, w2x, w2f)


def _run_stage_c(m2, xyz8, w2x, b2c, wm1, bm1c, wm2, bm2c, wl, blc):
    return pl.pallas_call(
        _stage_c,
        grid=(_B,),
        in_specs=[_batch_spec(32, _N), _batch_spec(8, _N),
                  _full_spec(64, 8), _full_spec(64, 1),
                  _full_spec(64, 64), _full_spec(64, 1),
                  _full_spec(64, 64), _full_spec(64, 1),
                  _full_spec(3, 64), _full_spec(3, 1)],
        out_specs=[_batch_spec(64, _N), _batch_spec(3, _N)],
        out_shape=[jax.ShapeDtypeStruct((_B, 64, _N), jnp.float32),
                   jax.ShapeDtypeStruct((_B, 3, _N), jnp.float32)],
    )(m2, xyz8, w2x, b2c, wm1, bm1c, wm2, bm2c, wl, blc)


@functools.cache
def _build_gather_max():
    mesh = plsc.VectorSubcoreMesh(core_axis_name="c", subcore_axis_name="s")
    return functools.partial(
        pl.kernel,
        mesh=mesh,
        compiler_params=pltpu.CompilerParams(
            use_tc_tiling_on_sc=False, needs_layout_passes=False),
        out_type=jax.ShapeDtypeStruct((_B * 32, _N), jnp.int32),
        scratch_types=[
            pltpu.VMEM((4, _N), jnp.int32),
            pltpu.VMEM((2, _K, _PB), jnp.int32),
            pltpu.VMEM((2, 4, _PB), jnp.int32),
            pltpu.SemaphoreType.DMA,
            pltpu.SemaphoreType.DMA,
            pltpu.SemaphoreType.DMA,
            pltpu.SemaphoreType.DMA,
        ],
    )(_gather_max_body)


def _gather_max(hp, knn_t):
    m = _build_gather_max()(hp.reshape(_B * 32, _N), knn_t)
    return m.reshape(_B, 32, _N)


def _gather_max_body(h_hbm, knn_hbm, m_hbm, table_v, knn_v, out_v,
                     sem_k0, sem_k1, sem_o0, sem_o1):
    cid = lax.axis_index("c")
    sid = lax.axis_index("s")
    wid = sid * 2 + cid
    b = wid // _GROUPS
    g = wid % _GROUPS
    row0 = b * 32 + g * 4

    pltpu.sync_copy(h_hbm.at[pl.ds(row0, 4), :], table_v)

    sem_k = (sem_k0, sem_k1)
    sem_o = (sem_o0, sem_o1)
    psplats = [jnp.full((16,), p, jnp.int32) for p in range(4)]

    def _knn_copy(blk, s):
        return pltpu.make_async_copy(
            knn_hbm.at[b, :, pl.ds(blk * _PB, _PB)], knn_v.at[s], sem_k[s])

    def _out_copy(blk, s):
        return pltpu.make_async_copy(
            out_v.at[s], m_hbm.at[pl.ds(row0, 4), pl.ds(blk * _PB, _PB)],
            sem_o[s])

    _knn_copy(0, 0).start()

    def outer(gi, carry):
        for s in range(2):
            blk = gi * 2 + s

            @pl.when(blk < _NB)
            def _process():
                @pl.when(blk + 1 < _NB)
                def _prefetch():
                    _knn_copy(blk + 1, 1 - s).start()

                _knn_copy(blk, s).wait()

                @pl.when(blk >= 2)
                def _drain():
                    _out_copy(blk - 2, s).wait()

                def sub(j, carry2):
                    p0 = j * 16
                    cols = [knn_v[s, k, pl.ds(p0, 16)] for k in range(_K)]
                    for p in range(4):
                        acc = plsc.bitcast(
                            plsc.load_gather(table_v, [psplats[p], cols[0]]),
                            jnp.int16)
                        for k in range(1, _K):
                            acc = jnp.maximum(acc, plsc.bitcast(
                                plsc.load_gather(table_v,
                                                 [psplats[p], cols[k]]),
                                jnp.int16))
                        out_v[s, p, pl.ds(p0, 16)] = plsc.bitcast(
                            acc, jnp.int32)
                    return carry2

                lax.fori_loop(0, _PB // 16, sub, 0)
                _out_copy(blk, s).start()
        return carry

    lax.fori_loop(0, (_NB + 1) // 2, outer, 0)
    _out_copy(_NB - 2, (_NB - 2) % 2).wait()
    _out_copy(_NB - 1, (_NB - 1) % 2).wait()


def kernel(xyz, feat, knn_indices, mask, W1, b1, W2, b2, Wm1, bm1, Wm2, bm2,
           Wl, bl):
    del mask
    knn = knn_indices.astype(jnp.int32)
    xyz8 = jnp.pad(xyz, ((0, 0), (0, 5), (0, 0)))
    w1x = jnp.pad(W1[:, :3], ((0, 0), (0, 5)))
    w2x = jnp.pad(W2[:, :3], ((0, 0), (0, 5)))

    knn_t = knn.transpose(0, 2, 1)
    h1 = _run_stage_a(xyz8, feat, w1x, W1[:, 3:])
    m1 = _gather_max(h1, knn_t)
    h2 = _run_stage_b(m1, xyz8, w1x, b1[:, None], w2x, W2[:, 3:])
    m2 = _gather_max(h2, knn_t)
    h, flow = _run_stage_c(m2, xyz8, w2x, b2[:, None], Wm1, bm1[:, None],
                           Wm2, bm2[:, None], Wl, bl[:, None])
    return (h, flow)


# speedup vs baseline: 1.0119x; 1.0119x over previous
import functools

import jax
import jax.numpy as jnp
from jax import lax
from jax.experimental import pallas as pl
from jax.experimental.pallas import tpu as pltpu
from jax.experimental.pallas import tpu_sc as plsc

_B, _N, _K = 4, 10000, 16
_GROUPS = 8
_PB = 400
_NB = _N // _PB


def _leaky(x):
    return jnp.where(x >= 0, x, 0.1 * x)


def _mm(w, x):
    return lax.dot_general(w, x, (((1,), (0,)), ((), ())))


def _order_map2(w):
    t = (w >> 15) & jnp.uint32(0x00010001)
    return w ^ (t * jnp.uint32(0x7FFF))


def _f16_bits(h):
    u = lax.bitcast_convert_type(h, jnp.uint32)
    sign = (u >> 16) & jnp.uint32(0x8000)
    mag = u & jnp.uint32(0x7FFFFFFF)
    r = (mag + jnp.uint32(0xFFF) + ((mag >> 13) & jnp.uint32(1))) >> 13
    m16 = jnp.maximum(
        lax.convert_element_type(r, jnp.int32) - jnp.int32(0x1C000),
        jnp.int32(0))
    return sign | lax.convert_element_type(m16, jnp.uint32)


def _pack(h):
    u = _f16_bits(h)
    return lax.bitcast_convert_type(_order_map2(u[:32] | (u[32:] << 16)),
                                    jnp.int32)


def _half_to_f32(b):
    sign32 = (b & jnp.uint32(0x8000)) << 16
    mag16 = b & jnp.uint32(0x7FFF)
    mag32 = (mag16 + jnp.uint32(0x1C000)) << 13
    mag32 = jnp.where(mag16 == 0, jnp.uint32(0), mag32)
    return lax.bitcast_convert_type(sign32 | mag32, jnp.float32)


def _unpack(mp):
    u = _order_map2(lax.bitcast_convert_type(mp, jnp.uint32))
    lo = _half_to_f32(u & jnp.uint32(0xFFFF))
    hi = _half_to_f32(u >> 16)
    return lax.concatenate([lo, hi], 0)


def _stage_a(xyz_ref, feat_ref, w1x_ref, w1f_ref, h1_ref):
    h1 = _mm(w1x_ref[...], xyz_ref[0]) + _mm(w1f_ref[...], feat_ref[0])
    h1_ref[0] = _pack(h1)


def _stage_b(m1_ref, xyz_ref, w1x_ref, b1_ref, w2x_ref, w2f_ref, h2_ref):
    f1 = _leaky(_unpack(m1_ref[0]) - _mm(w1x_ref[...], xyz_ref[0])
                + b1_ref[...])
    h2_ref[0] = _pack(_mm(w2x_ref[...], xyz_ref[0]) + _mm(w2f_ref[...], f1))


def _stage_c(m2_ref, xyz_ref, w2x_ref, b2_ref, wm1_ref, bm1_ref, wm2_ref,
             bm2_ref, wl_ref, bl_ref, h_ref, flow_ref):
    f2 = _leaky(_unpack(m2_ref[0]) - _mm(w2x_ref[...], xyz_ref[0])
                + b2_ref[...])
    h1 = _leaky(_mm(wm1_ref[...], f2) + bm1_ref[...])
    h = _leaky(_mm(wm2_ref[...], h1) + bm2_ref[...])
    h_ref[0] = h
    flow_ref[0] = _mm(wl_ref[...], h) + bl_ref[...]


def _batch_spec(c, n):
    return pl.BlockSpec((1, c, n), lambda b: (b, 0, 0))


def _full_spec(*shape):
    return pl.BlockSpec(shape, lambda b: tuple(0 for _ in shape))


def _run_stage_a(xyz8, feat, w1x, w1f):
    return pl.pallas_call(
        _stage_a,
        grid=(_B,),
        in_specs=[_batch_spec(8, _N), _batch_spec(128, _N),
                  _full_spec(64, 8), _full_spec(64, 128)],
        out_specs=_batch_spec(32, _N),
        out_shape=jax.ShapeDtypeStruct((_B, 32, _N), jnp.int32),
    )(xyz8, feat, w1x, w1f)


def _run_stage_b(m1, xyz8, w1x, b1c, w2x, w2f):
    return pl.pallas_call(
        _stage_b,
        grid=(_B,),
        in_specs=[_batch_spec(32, _N), _batch_spec(8, _N),
                  _full_spec(64, 8), _full_spec(64, 1),
                  _full_spec(64, 8), _full_spec(64, 64)],
        out_specs=_batch_spec(32, _N),
        out_shape=jax.ShapeDtypeStruct((_B, 32, _N), jnp.int32),
    )(m1, xyz8, w1x, b1c, w2x, w2f)


def _run_stage_c(m2, xyz8, w2x, b2c, wm1, bm1c, wm2, bm2c, wl, blc):
    return pl.pallas_call(
        _stage_c,
        grid=(_B,),
        in_specs=[_batch_spec(32, _N), _batch_spec(8, _N),
                  _full_spec(64, 8), _full_spec(64, 1),
                  _full_spec(64, 64), _full_spec(64, 1),
                  _full_spec(64, 64), _full_spec(64, 1),
                  _full_spec(3, 64), _full_spec(3, 1)],
        out_specs=[_batch_spec(64, _N), _batch_spec(3, _N)],
        out_shape=[jax.ShapeDtypeStruct((_B, 64, _N), jnp.float32),
                   jax.ShapeDtypeStruct((_B, 3, _N), jnp.float32)],
    )(m2, xyz8, w2x, b2c, wm1, bm1c, wm2, bm2c, wl, blc)


@functools.cache
def _build_gather_max():
    mesh = plsc.VectorSubcoreMesh(core_axis_name="c", subcore_axis_name="s")
    return functools.partial(
        pl.kernel,
        mesh=mesh,
        compiler_params=pltpu.CompilerParams(
            use_tc_tiling_on_sc=False, needs_layout_passes=False),
        out_type=jax.ShapeDtypeStruct((_B * 32, _N), jnp.int32),
        scratch_types=[
            pltpu.VMEM((4, _N), jnp.int32),
            pltpu.VMEM((2, _K, _PB), jnp.int32),
            pltpu.VMEM((2, 4, _PB), jnp.int32),
            pltpu.SemaphoreType.DMA,
            pltpu.SemaphoreType.DMA,
            pltpu.SemaphoreType.DMA,
            pltpu.SemaphoreType.DMA,
        ],
    )(_gather_max_body)


def _gather_max(hp, knn_t):
    m = _build_gather_max()(hp.reshape(_B * 32, _N), knn_t)
    return m.reshape(_B, 32, _N)


def _gather_max_body(h_hbm, knn_hbm, m_hbm, table_v, knn_v, out_v,
                     sem_k0, sem_k1, sem_o0, sem_o1):
    cid = lax.axis_index("c")
    sid = lax.axis_index("s")
    wid = sid * 2 + cid
    b = wid // _GROUPS
    g = wid % _GROUPS
    row0 = b * 32 + g * 4

    pltpu.sync_copy(h_hbm.at[pl.ds(row0, 4), :], table_v)

    sem_k = (sem_k0, sem_k1)
    sem_o = (sem_o0, sem_o1)
    psplats = [jnp.full((16,), p, jnp.int32) for p in range(4)]

    def _knn_copy(blk, s):
        return pltpu.make_async_copy(
            knn_hbm.at[b, :, pl.ds(blk * _PB, _PB)], knn_v.at[s], sem_k[s])

    def _out_copy(blk, s):
        return pltpu.make_async_copy(
            out_v.at[s], m_hbm.at[pl.ds(row0, 4), pl.ds(blk * _PB, _PB)],
            sem_o[s])

    _knn_copy(0, 0).start()

    def outer(gi, carry):
        for s in range(2):
            blk = gi * 2 + s

            @pl.when(blk < _NB)
            def _process():
                @pl.when(blk + 1 < _NB)
                def _prefetch():
                    _knn_copy(blk + 1, 1 - s).start()

                _knn_copy(blk, s).wait()

                @pl.when(blk >= 2)
                def _drain():
                    _out_copy(blk - 2, s).wait()

                def sub(j, carry2):
                    p0 = j * 16
                    cols = [knn_v[s, k, pl.ds(p0, 16)] for k in range(_K)]
                    for p in range(4):
                        acc = plsc.bitcast(
                            plsc.load_gather(table_v, [psplats[p], cols[0]]),
                            jnp.int16)
                        for k in range(1, _K):
                            acc = jnp.maximum(acc, plsc.bitcast(
                                plsc.load_gather(table_v,
                                                 [psplats[p], cols[k]]),
                                jnp.int16))
                        out_v[s, p, pl.ds(p0, 16)] = plsc.bitcast(
                            acc, jnp.int32)
                    return carry2

                lax.fori_loop(0, _PB // 16, sub, 0)
                _out_copy(blk, s).start()
        return carry

    lax.fori_loop(0, (_NB + 1) // 2, outer, 0)
    _out_copy(_NB - 2, (_NB - 2) % 2).wait()
    _out_copy(_NB - 1, (_NB - 1) % 2).wait()


def kernel(xyz, feat, knn_indices, mask, W1, b1, W2, b2, Wm1, bm1, Wm2, bm2,
           Wl, bl):
    del mask
    knn = knn_indices.astype(jnp.int32)
    xyz8 = jnp.pad(xyz, ((0, 0), (0, 5), (0, 0)))
    w1x = jnp.pad(W1[:, :3], ((0, 0), (0, 5)))
    w2x = jnp.pad(W2[:, :3], ((0, 0), (0, 5)))

    knn_t = knn.transpose(0, 2, 1)
    h1 = _run_stage_a(xyz8, feat, w1x, W1[:, 3:])
    m1 = _gather_max(h1, knn_t)
    h2 = _run_stage_b(m1, xyz8, w1x, b1[:, None], w2x, W2[:, 3:])
    m2 = _gather_max(h2, knn_t)
    h, flow = _run_stage_c(m2, xyz8, w2x, b2[:, None], Wm1, bm1[:, None],
                           Wm2, bm2[:, None], Wl, bl[:, None])
    return (h, flow)


# speedup vs baseline: 1.0556x; 1.0432x over previous
import functools

import jax
import jax.numpy as jnp
from jax import lax
from jax.experimental import pallas as pl
from jax.experimental.pallas import tpu as pltpu
from jax.experimental.pallas import tpu_sc as plsc

_B, _N, _K = 4, 10000, 16
_GROUPS = 8
_PB = 400
_NB = _N // _PB


def _leaky(x):
    return jnp.where(x >= 0, x, 0.1 * x)


def _mm(w, x):
    return lax.dot_general(w, x, (((1,), (0,)), ((), ())))


def _order_map2(w):
    t = (w >> 15) & jnp.uint32(0x00010001)
    return w ^ (t * jnp.uint32(0x7FFF))


def _f16_bits(h):
    u = lax.bitcast_convert_type(h, jnp.uint32)
    sign = (u >> 16) & jnp.uint32(0x8000)
    mag = u & jnp.uint32(0x7FFFFFFF)
    r = (mag + jnp.uint32(0xFFF) + ((mag >> 13) & jnp.uint32(1))) >> 13
    m16 = jnp.maximum(
        lax.convert_element_type(r, jnp.int32) - jnp.int32(0x1C000),
        jnp.int32(0))
    return sign | lax.convert_element_type(m16, jnp.uint32)


def _pack(h):
    u = _f16_bits(h)
    return lax.bitcast_convert_type(_order_map2(u[:32] | (u[32:] << 16)),
                                    jnp.int32)


def _half_to_f32(b):
    sign32 = (b & jnp.uint32(0x8000)) << 16
    mag16 = b & jnp.uint32(0x7FFF)
    mag32 = (mag16 + jnp.uint32(0x1C000)) << 13
    mag32 = jnp.where(mag16 == 0, jnp.uint32(0), mag32)
    return lax.bitcast_convert_type(sign32 | mag32, jnp.float32)


def _unpack(mp):
    u = _order_map2(lax.bitcast_convert_type(mp, jnp.uint32))
    lo = _half_to_f32(u & jnp.uint32(0xFFFF))
    hi = _half_to_f32(u >> 16)
    return lax.concatenate([lo, hi], 0)


def _stage_a(xyz_ref, feat_ref, w1x_ref, w1f_ref, h1_ref):
    h1 = _mm(w1x_ref[...], xyz_ref[0]) + _mm(w1f_ref[...], feat_ref[0])
    h1_ref[0] = _pack(h1)


def _stage_b(m1_ref, xyz_ref, w1x_ref, b1_ref, w2x_ref, w2f_ref, h2_ref):
    f1 = _leaky(_unpack(m1_ref[0]) - _mm(w1x_ref[...], xyz_ref[0])
                + b1_ref[...])
    h2_ref[0] = _pack(_mm(w2x_ref[...], xyz_ref[0]) + _mm(w2f_ref[...], f1))


def _stage_c(m2_ref, xyz_ref, w2x_ref, b2_ref, wm1_ref, bm1_ref, wm2_ref,
             bm2_ref, wl_ref, bl_ref, h_ref, flow_ref):
    f2 = _leaky(_unpack(m2_ref[0]) - _mm(w2x_ref[...], xyz_ref[0])
                + b2_ref[...])
    h1 = _leaky(_mm(wm1_ref[...], f2) + bm1_ref[...])
    h = _leaky(_mm(wm2_ref[...], h1) + bm2_ref[...])
    h_ref[0] = h
    flow_ref[0] = _mm(wl_ref[...], h) + bl_ref[...]


def _batch_spec(c, n):
    return pl.BlockSpec((1, c, n), lambda b: (b, 0, 0))


def _full_spec(*shape):
    return pl.BlockSpec(shape, lambda b: tuple(0 for _ in shape))


def _run_stage_a(xyz8, feat, w1x, w1f):
    return pl.pallas_call(
        _stage_a,
        grid=(_B,),
        in_specs=[_batch_spec(8, _N), _batch_spec(128, _N),
                  _full_spec(64, 8), _full_spec(64, 128)],
        out_specs=_batch_spec(32, _N),
        out_shape=jax.ShapeDtypeStruct((_B, 32, _N), jnp.int32),
    )(xyz8, feat, w1x, w1f)


def _run_stage_b(m1, xyz8, w1x, b1c, w2x, w2f):
    return pl.pallas_call(
        _stage_b,
        grid=(_B,),
        in_specs=[_batch_spec(32, _N), _batch_spec(8, _N),
                  _full_spec(64, 8), _full_spec(64, 1),
                  _full_spec(64, 8), _full_spec(64, 64)],
        out_specs=_batch_spec(32, _N),
        out_shape=jax.ShapeDtypeStruct((_B, 32, _N), jnp.int32),
    )(m1, xyz8, w1x, b1c, w2x, w2f)


def _run_stage_c(m2, xyz8, w2x, b2c, wm1, bm1c, wm2, bm2c, wl, blc):
    return pl.pallas_call(
        _stage_c,
        grid=(_B,),
        in_specs=[_batch_spec(32, _N), _batch_spec(8, _N),
                  _full_spec(64, 8), _full_spec(64, 1),
                  _full_spec(64, 64), _full_spec(64, 1),
                  _full_spec(64, 64), _full_spec(64, 1),
                  _full_spec(3, 64), _full_spec(3, 1)],
        out_specs=[_batch_spec(64, _N), _batch_spec(3, _N)],
        out_shape=[jax.ShapeDtypeStruct((_B, 64, _N), jnp.float32),
                   jax.ShapeDtypeStruct((_B, 3, _N), jnp.float32)],
    )(m2, xyz8, w2x, b2c, wm1, bm1c, wm2, bm2c, wl, blc)


@functools.cache
def _build_gather_max():
    mesh = plsc.VectorSubcoreMesh(core_axis_name="c", subcore_axis_name="s")
    return functools.partial(
        pl.kernel,
        mesh=mesh,
        compiler_params=pltpu.CompilerParams(
            use_tc_tiling_on_sc=False, needs_layout_passes=False),
        out_type=jax.ShapeDtypeStruct((_B * 32, _N), jnp.int32),
        scratch_types=[
            pltpu.VMEM((4, _N), jnp.int32),
            pltpu.VMEM((2, _K, _PB), jnp.int32),
            pltpu.VMEM((2, 4, _PB), jnp.int32),
            pltpu.SemaphoreType.DMA,
            pltpu.SemaphoreType.DMA,
            pltpu.SemaphoreType.DMA,
            pltpu.SemaphoreType.DMA,
        ],
    )(_gather_max_body)


def _gather_max(hp, knn_t):
    m = _build_gather_max()(hp.reshape(_B * 32, _N), knn_t)
    return m.reshape(_B, 32, _N)


def _gather_max_body(h_hbm, knn_hbm, m_hbm, table_v, knn_v, out_v,
                     sem_k0, sem_k1, sem_o0, sem_o1):
    cid = lax.axis_index("c")
    sid = lax.axis_index("s")
    wid = sid * 2 + cid
    b = wid // _GROUPS
    g = wid % _GROUPS
    row0 = b * 32 + g * 4

    pltpu.sync_copy(h_hbm.at[pl.ds(row0, 4), :], table_v)

    sem_k = (sem_k0, sem_k1)
    sem_o = (sem_o0, sem_o1)
    psplats = [jnp.full((16,), p, jnp.int32) for p in range(4)]

    def _knn_copy(blk, s):
        return pltpu.make_async_copy(
            knn_hbm.at[b, :, pl.ds(blk * _PB, _PB)], knn_v.at[s], sem_k[s])

    def _out_copy(blk, s):
        return pltpu.make_async_copy(
            out_v.at[s], m_hbm.at[pl.ds(row0, 4), pl.ds(blk * _PB, _PB)],
            sem_o[s])

    _knn_copy(0, 0).start()

    def outer(gi, carry):
        for s in range(2):
            blk = gi * 2 + s

            @pl.when(blk < _NB)
            def _process():
                @pl.when(blk + 1 < _NB)
                def _prefetch():
                    _knn_copy(blk + 1, 1 - s).start()

                _knn_copy(blk, s).wait()

                @pl.when(blk >= 2)
                def _drain():
                    _out_copy(blk - 2, s).wait()

                def sub(j, carry2):
                    for u in range(5):
                        p0 = j * 80 + u * 16
                        cols = [knn_v[s, k, pl.ds(p0, 16)]
                                for k in range(_K)]
                        for p in range(4):
                            g = [plsc.bitcast(
                                    plsc.load_gather(table_v,
                                                     [psplats[p], cols[k]]),
                                    jnp.int16) for k in range(_K)]
                            while len(g) > 1:
                                g = [jnp.maximum(g[i], g[i + 1])
                                     for i in range(0, len(g), 2)]
                            out_v[s, p, pl.ds(p0, 16)] = plsc.bitcast(
                                g[0], jnp.int32)
                    return carry2

                lax.fori_loop(0, _PB // 80, sub, 0)
                _out_copy(blk, s).start()
        return carry

    lax.fori_loop(0, (_NB + 1) // 2, outer, 0)
    _out_copy(_NB - 2, (_NB - 2) % 2).wait()
    _out_copy(_NB - 1, (_NB - 1) % 2).wait()


def kernel(xyz, feat, knn_indices, mask, W1, b1, W2, b2, Wm1, bm1, Wm2, bm2,
           Wl, bl):
    del mask
    knn = knn_indices.astype(jnp.int32)
    xyz8 = jnp.pad(xyz, ((0, 0), (0, 5), (0, 0)))
    w1x = jnp.pad(W1[:, :3], ((0, 0), (0, 5)))
    w2x = jnp.pad(W2[:, :3], ((0, 0), (0, 5)))

    knn_t = knn.transpose(0, 2, 1)
    h1 = _run_stage_a(xyz8, feat, w1x, W1[:, 3:])
    m1 = _gather_max(h1, knn_t)
    h2 = _run_stage_b(m1, xyz8, w1x, b1[:, None], w2x, W2[:, 3:])
    m2 = _gather_max(h2, knn_t)
    h, flow = _run_stage_c(m2, xyz8, w2x, b2[:, None], Wm1, bm1[:, None],
                           Wm2, bm2[:, None], Wl, bl[:, None])
    return (h, flow)
